# Initial kernel scaffold; baseline (speedup 1.0000x reference)
import jax
import jax.numpy as jnp
from jax.experimental import pallas as pl


def kernel(entity_indices, entity_embeddings):
    raise NotImplementedError("write your pallas kernel here")


# speedup vs baseline: 1.4597x; 1.4597x over previous
import functools

import jax
import jax.numpy as jnp
from jax import lax
from jax.experimental import pallas as pl
from jax.experimental.pallas import tpu as pltpu
from jax.experimental.pallas import tpu_sc as plsc

_B, _S = 4096, 200
_N = _B * _S
_D = 32
_NC, _NS = 2, 16
_NW = _NC * _NS
_PER_W = _N // _NW
_CHUNK = 1024
_NCHUNK = _PER_W // _CHUNK

_mesh = plsc.VectorSubcoreMesh(core_axis_name="c", subcore_axis_name="s")


@functools.partial(
    pl.kernel,
    mesh=_mesh,
    out_type=jax.ShapeDtypeStruct((_N, _D), jnp.float32),
    scratch_types=[
        pltpu.VMEM((_CHUNK,), jnp.int32),
        pltpu.VMEM((_CHUNK, _D), jnp.float32),
        pltpu.SemaphoreType.DMA,
    ],
    compiler_params=pltpu.CompilerParams(use_tc_tiling_on_sc=False),
)
def _gather_kernel(idx_hbm, table_hbm, out_hbm, idx_v, rows_v, sem):
    wid = lax.axis_index("s") * _NC + lax.axis_index("c")
    base = wid * _PER_W

    def chunk_body(i, carry):
        off = base + i * _CHUNK
        pltpu.sync_copy(idx_hbm.at[pl.ds(off, _CHUNK)], idx_v)
        pltpu.async_copy(table_hbm.at[idx_v], rows_v, sem).wait()
        pltpu.sync_copy(rows_v, out_hbm.at[pl.ds(off, _CHUNK)])
        return carry

    lax.fori_loop(0, _NCHUNK, chunk_body, 0)


def kernel(entity_indices, entity_embeddings):
    flat_idx = entity_indices.reshape(_N)
    out = _gather_kernel(flat_idx, entity_embeddings)
    return out.reshape(_B, _S, _D)


# speedup vs baseline: 1.4949x; 1.0241x over previous
import functools

import jax
import jax.numpy as jnp
from jax import lax
from jax.experimental import pallas as pl
from jax.experimental.pallas import tpu as pltpu
from jax.experimental.pallas import tpu_sc as plsc

_B, _S = 4096, 200
_N = _B * _S
_D = 32
_NC, _NS = 2, 16
_NW = _NC * _NS
_PER_W = _N // _NW
_CHUNK = 1280
_NCHUNK = _PER_W // _CHUNK

_mesh = plsc.VectorSubcoreMesh(core_axis_name="c", subcore_axis_name="s")


@functools.partial(
    pl.kernel,
    mesh=_mesh,
    out_type=jax.ShapeDtypeStruct((_N, _D), jnp.float32),
    scratch_types=[
        pltpu.VMEM((_PER_W,), jnp.int32),
        pltpu.VMEM((_CHUNK, _D), jnp.float32),
        pltpu.VMEM((_CHUNK, _D), jnp.float32),
        pltpu.SemaphoreType.DMA,
        pltpu.SemaphoreType.DMA,
        pltpu.SemaphoreType.DMA,
        pltpu.SemaphoreType.DMA,
    ],
    compiler_params=pltpu.CompilerParams(use_tc_tiling_on_sc=False),
)
def _gather_kernel(idx_hbm, table_hbm, out_hbm, idx_all, rows0, rows1,
                   gsem0, gsem1, wsem0, wsem1):
    wid = lax.axis_index("s") * _NC + lax.axis_index("c")
    base = wid * _PER_W

    rows = (rows0, rows1)
    gsem = (gsem0, gsem1)
    wsem = (wsem0, wsem1)

    pltpu.sync_copy(idx_hbm.at[pl.ds(base, _PER_W)], idx_all)

    def start_gather(i, b):
        idx_view = idx_all.at[pl.ds(i * _CHUNK, _CHUNK)]
        pltpu.async_copy(table_hbm.at[idx_view], rows[b], gsem[b])

    def start_write(i, b):
        pltpu.async_copy(rows[b], out_hbm.at[pl.ds(base + i * _CHUNK, _CHUNK)],
                         wsem[b])

    def wait_gather(b):
        pltpu.make_async_copy(table_hbm.at[idx_all.at[pl.ds(0, _CHUNK)]],
                              rows[b], gsem[b]).wait()

    def wait_write(b):
        pltpu.make_async_copy(rows[b], out_hbm.at[pl.ds(0, _CHUNK)],
                              wsem[b]).wait()

    start_gather(0, 0)

    def outer(g, carry):
        for b in range(2):
            i = g + b
            wait_gather(b)
            start_write(i, b)

            @pl.when(i >= 1)
            def _():
                wait_write(b ^ 1)

            @pl.when(i + 1 < _NCHUNK)
            def _():
                start_gather(i + 1, b ^ 1)

        return carry

    lax.fori_loop(0, _NCHUNK // 2, lambda t, c: outer(t * 2, c), 0)
    wait_write((_NCHUNK - 1) & 1)


def kernel(entity_indices, entity_embeddings):
    flat_idx = entity_indices.reshape(_N)
    out = _gather_kernel(flat_idx, entity_embeddings)
    return out.reshape(_B, _S, _D)


# speedup vs baseline: 1.5680x; 1.0489x over previous
import functools

import jax
import jax.numpy as jnp
from jax import lax
from jax.experimental import pallas as pl
from jax.experimental.pallas import tpu as pltpu
from jax.experimental.pallas import tpu_sc as plsc

_B, _S = 4096, 200
_D = 32
_NC, _NS = 2, 16
_NW = _NC * _NS
_CHUNK = 1024
_CPS = _B // _CHUNK
_NUNIT = _S * _CPS
_PER_W = _NUNIT // _NW

_mesh = plsc.VectorSubcoreMesh(core_axis_name="c", subcore_axis_name="s")


@functools.partial(
    pl.kernel,
    mesh=_mesh,
    out_type=jax.ShapeDtypeStruct((_S, _B, _D), jnp.float32),
    scratch_types=[
        pltpu.VMEM((2, _CHUNK), jnp.int32),
        pltpu.VMEM((_CHUNK, _D), jnp.float32),
        pltpu.VMEM((_CHUNK, _D), jnp.float32),
        pltpu.SemaphoreType.DMA,
        pltpu.SemaphoreType.DMA,
        pltpu.SemaphoreType.DMA,
        pltpu.SemaphoreType.DMA,
        pltpu.SemaphoreType.DMA,
        pltpu.SemaphoreType.DMA,
    ],
    compiler_params=pltpu.CompilerParams(use_tc_tiling_on_sc=False),
)
def _gather_kernel(idx_hbm, table_hbm, out_hbm, idx_v, rows0, rows1,
                   isem0, isem1, gsem0, gsem1, wsem0, wsem1):
    wid = lax.axis_index("s") * _NC + lax.axis_index("c")
    u0 = wid * _PER_W

    rows = (rows0, rows1)
    isem = (isem0, isem1)
    gsem = (gsem0, gsem1)
    wsem = (wsem0, wsem1)

    def idx_src(u):
        s, c = u // _CPS, u % _CPS
        return idx_hbm.at[s, pl.ds(c * _CHUNK, _CHUNK)]

    def start_idx(j, b):
        pltpu.async_copy(idx_src(u0 + j), idx_v.at[b], isem[b])

    def gather(j, b):
        pltpu.make_async_copy(idx_src(u0), idx_v.at[b], isem[b]).wait()
        pltpu.async_copy(table_hbm.at[idx_v.at[b]], rows[b], gsem[b])

    def start_write(j, b):
        u = u0 + j
        s, c = u // _CPS, u % _CPS
        pltpu.make_async_copy(table_hbm.at[idx_v.at[b]], rows[b],
                              gsem[b]).wait()
        pltpu.async_copy(rows[b], out_hbm.at[s, pl.ds(c * _CHUNK, _CHUNK)],
                         wsem[b])

    def wait_write(b):
        pltpu.make_async_copy(rows[b], out_hbm.at[0, pl.ds(0, _CHUNK)],
                              wsem[b]).wait()

    start_idx(0, 0)
    gather(0, 0)
    start_idx(1, 1)

    def body(j, carry):
        b = lax.rem(j, 2)

        @pl.when(b == 0)
        def _():
            start_write(j, 0)

            @pl.when(j + 1 < _PER_W)
            def _():
                gather(j + 1, 1)

            @pl.when(j + 2 < _PER_W)
            def _():
                wait_write(0)
                start_idx(j + 2, 0)

        @pl.when(b == 1)
        def _():
            start_write(j, 1)

            @pl.when(j + 1 < _PER_W)
            def _():
                gather(j + 1, 0)

            @pl.when(j + 2 < _PER_W)
            def _():
                wait_write(1)
                start_idx(j + 2, 1)

        return carry

    lax.fori_loop(0, _PER_W, body, 0)
    wait_write((_PER_W - 1) % 2)
    wait_write(_PER_W % 2)


def kernel(entity_indices, entity_embeddings):
    idx_t = jnp.transpose(entity_indices)
    out = _gather_kernel(idx_t, entity_embeddings)
    return jnp.transpose(out, (1, 0, 2))
